# baseline (speedup 1.0000x reference)
import functools

import jax
import jax.numpy as jnp
from jax import lax
from jax.experimental import pallas as pl
from jax.experimental.pallas import tpu as pltpu
from jax.experimental.pallas import tpu_sc as plsc

B = 16384
S = 200
D = 16
N = B * S
NUM_ACTIONS = 1000

OTAB_ROWS = 1000000 // 8
ATAB_ROWS = 128

NC = 2
NS = 16
NW = NC * NS

TOK_PER_W = N // NW
CHUNK = 4096
SUB = 128
NSUB = CHUNK // SUB
NG = 3
NO = 2
ITERS = TOK_PER_W // CHUNK


def _make_kernel():
  mesh = plsc.VectorSubcoreMesh(core_axis_name="c", subcore_axis_name="s")

  @functools.partial(
      pl.kernel,
      mesh=mesh,
      out_type=jax.ShapeDtypeStruct((N // 4, 128), jnp.float32),
      scratch_types=(
          [pltpu.VMEM((CHUNK,), jnp.int32)] * 2 +
          [pltpu.VMEM((CHUNK // 128, 128), jnp.int32)] +
          [pltpu.VMEM((ATAB_ROWS, 128), jnp.float32)] +
          [pltpu.VMEM((SUB, 128), jnp.float32)] * NG +
          [pltpu.VMEM((SUB // 4, 128), jnp.float32)] * NO +
          [pltpu.SemaphoreType.DMA] * (NG + NO)
      ),
      compiler_params=pltpu.CompilerParams(needs_layout_passes=False),
  )
  def enc(aidx_hbm, oidx_hbm, atab_hbm, otab_hbm, out_hbm,
          aidx_v, oidx_v, coarse_v, atab_v, *bufs):
    gbufs = bufs[:NG]
    outvs = bufs[NG:NG + NO]
    gsems = bufs[NG + NO:NG + NO + NG]
    osems = bufs[NG + NO + NG:]
    wid = lax.axis_index("s") * NC + lax.axis_index("c")
    tok_base = wid * TOK_PER_W

    pltpu.sync_copy(atab_hbm, atab_v)
    iota = lax.iota(jnp.int32, 16)

    def fire_gather(s):
      g = s % NG
      return pltpu.async_copy(otab_hbm.at[coarse_v.at[s]],
                              gbufs[g], gsems[g])

    def body(it, carry):
      tok0 = tok_base + it * CHUNK
      pltpu.sync_copy(aidx_hbm.at[pl.ds(tok0, CHUNK)], aidx_v)
      pltpu.sync_copy(oidx_hbm.at[pl.ds(tok0, CHUNK)], oidx_v)
      for l in range(CHUNK // 16):
        v = oidx_v[pl.ds(l * 16, 16)]
        coarse_v[l // 8, pl.ds((l % 8) * 16, 16)] = v >> 3

      pend_g = [fire_gather(0), fire_gather(1)]
      pend_o = [None] * NO
      for s in range(NSUB):
        gb, ov = gbufs[s % NG], outvs[s % NO]
        if s + 2 < NSUB:
          pend_g.append(fire_gather(s + 2))
        pend_g.pop(0).wait()
        if pend_o[s % NO] is not None:
          pend_o[s % NO].wait()
          pend_o[s % NO] = None

        def _select(k, carry2):
          off = s * SUB + k * 16
          avec = aidx_v[pl.ds(off, 16)]
          ovec = oidx_v[pl.ds(off, 16)]
          for u in range(16):
            tloc = k * 16 + u
            a = avec[u]
            o = ovec[u]
            orow = lax.shift_right_logical(tloc, 2)
            ocol = (tloc & 3) * 32
            av = atab_v[a >> 3, pl.ds((a & 7) * D, D)]
            gv = gb[tloc, pl.ds((o & 7) * D, D)]
            ov[orow, pl.ds(ocol, D)] = av
            ov[orow, pl.ds(ocol + D, D)] = gv
          return carry2

        lax.fori_loop(0, SUB // 16, _select, 0)
        out_row0 = pl.multiple_of(tok0 // 4 + s * (SUB // 4), 8)
        pend_o[s % NO] = pltpu.async_copy(
            ov, out_hbm.at[pl.ds(out_row0, SUB // 4)], osems[s % NO])
      for d in pend_o:
        if d is not None:
          d.wait()
      return carry

    lax.fori_loop(0, ITERS, body, 0)

  return enc


_ENC = _make_kernel()


def kernel(action_idx, object_idx, action_table, object_table):
  aidx = action_idx.reshape(N).astype(jnp.int32)
  oidx = object_idx.reshape(N).astype(jnp.int32)
  atab = jnp.concatenate(
      [action_table,
       jnp.zeros((ATAB_ROWS * 8 - NUM_ACTIONS, D), jnp.float32)],
      axis=0).reshape(ATAB_ROWS, 128)
  otab = object_table.reshape(OTAB_ROWS, 128)
  out = _ENC(aidx, oidx, atab, otab)
  return out.reshape(B, S, 2 * D)
